# baseline (speedup 1.0000x reference)
import functools

import jax
import jax.numpy as jnp
from jax import lax
from jax.experimental import pallas as pl
from jax.experimental.pallas import tpu as pltpu
from jax.experimental.pallas import tpu_sc as plsc

E = 8
T = 2048
H = 1024
F = 2048

M = 128
NT = (T + E * (M - 1) + M - 1) // M
P = NT * M
K = 2
FK = F // K

_NC = 2
_NS = 16
_NW = _NC * _NS
_TPW = T // _NW


_C = 128
_NCH = T // _C


def _router_body(probs_ref, pos_ref, bk_ref):
    probs = probs_ref[...]
    lane = lax.broadcasted_iota(jnp.int32, (T, 128), 1)
    masked = jnp.where(lane < E, probs, -1.0)
    mx = jnp.max(masked, axis=1, keepdims=True)
    eid = jnp.min(jnp.where(masked == mx, lane, 127), axis=1, keepdims=True)

    onehot = (lane == eid).astype(jnp.float32)

    li = lax.broadcasted_iota(jnp.int32, (_C, _C), 0)
    lj = lax.broadcasted_iota(jnp.int32, (_C, _C), 1)
    ltri = (lj < li).astype(jnp.float32)
    pieces = []
    tots = []
    for c in range(_NCH):
        chunk = lax.slice(onehot, (c * _C, 0), ((c + 1) * _C, 128))
        pieces.append(jnp.dot(ltri, chunk, preferred_element_type=jnp.float32))
        tots.append(jnp.sum(chunk, axis=0, keepdims=True))
    cum_within = jnp.concatenate(pieces, axis=0)
    chunk_tot = jnp.concatenate(tots, axis=0)

    ci = lax.broadcasted_iota(jnp.int32, (_NCH, _NCH), 0)
    cj = lax.broadcasted_iota(jnp.int32, (_NCH, _NCH), 1)
    ltri_c = (cj < ci).astype(jnp.float32)
    offsets = jnp.dot(ltri_c, chunk_tot, preferred_element_type=jnp.float32)

    bi = lax.broadcasted_iota(jnp.int32, (T, _NCH), 0)
    bj = lax.broadcasted_iota(jnp.int32, (T, _NCH), 1)
    expand = ((bi // _C) == bj).astype(jnp.float32)
    cum = cum_within + jnp.dot(expand, offsets,
                               preferred_element_type=jnp.float32,
                               precision=lax.Precision.HIGHEST)
    rank = jnp.sum(cum * onehot, axis=1, keepdims=True)

    counts = jnp.sum(chunk_tot, axis=0, keepdims=True)
    ntiles = jnp.ceil(counts * (1.0 / M))
    ui = lax.broadcasted_iota(jnp.int32, (128, 128), 0)
    uj = lax.broadcasted_iota(jnp.int32, (128, 128), 1)
    utri = (ui < uj).astype(jnp.float32)
    start = jnp.dot(ntiles, utri, preferred_element_type=jnp.float32)
    po = start * M

    pos = rank + jnp.sum(onehot * po, axis=1, keepdims=True)
    pos_ref[...] = pos.astype(jnp.int32)

    lane_row = lax.broadcasted_iota(jnp.int32, (1, 128), 1)
    n_active = jnp.sum(ntiles).astype(jnp.int32)
    active_e = (counts > 0) & (lane_row < E)
    act_f = active_e.astype(jnp.float32)
    last_e = jnp.max(jnp.where(active_e, lane_row, 0))
    rank_e = jnp.dot(act_f, utri, preferred_element_type=jnp.float32)

    ti = lax.broadcasted_iota(jnp.int32, (NT, 128), 0).astype(jnp.float32)
    lane_t = lax.broadcasted_iota(jnp.int32, (NT, 128), 1)
    in_grp = (ti >= start) & (ti < start + ntiles) & (lane_t < E)
    te_raw = jnp.sum(jnp.where(in_grp, lane_t, 0), axis=1, keepdims=True)
    ti_col = lax.broadcasted_iota(jnp.int32, (NT, 1), 0)
    is_act = ti_col < n_active
    te_col = jnp.where(is_act, te_raw, last_e)
    xi_col = jnp.where(is_act, ti_col, n_active - 1)
    act_col = is_act.astype(jnp.int32)

    d_col = jnp.sum(jnp.where(in_grp, rank_e, 0.0), axis=1, keepdims=True)
    slot_col = lax.rem(d_col.astype(jnp.int32), 2)
    first_col = jnp.sum(jnp.where(in_grp & (ti == start), 1, 0),
                        axis=1, keepdims=True)
    last_col = jnp.sum(jnp.where(in_grp & (ti == start + ntiles - 1.0), 1, 0),
                       axis=1, keepdims=True)
    fetch_ind = (rank_e == d_col + 2.0) & active_e & (lane_t < E)
    fetch_col = jnp.sum(jnp.where(fetch_ind, lane_t, 0), axis=1, keepdims=True)
    hasf_col = jnp.sum(jnp.where(fetch_ind, 1, 0), axis=1, keepdims=True)
    sec_ind = (rank_e == 1.0) & active_e
    sec_e = jnp.max(jnp.where(sec_ind, lane_row, 0))
    has_sec = jnp.max(jnp.where(sec_ind, 1, 0))
    sec_col = jnp.full_like(te_col, 0) + sec_e
    hass_col = jnp.full_like(te_col, 0) + has_sec

    bk_ref[...] = jnp.concatenate(
        [te_col, xi_col, act_col, slot_col, first_col, last_col,
         fetch_col, hasf_col, sec_col, hass_col], axis=1)


def _router(probs_pad):
    return pl.pallas_call(
        _router_body,
        out_shape=(
            jax.ShapeDtypeStruct((T, 1), jnp.int32),
            jax.ShapeDtypeStruct((NT, 10), jnp.int32),
        ),
    )(probs_pad)


@functools.cache
def _make_dispatch():
    mesh = plsc.VectorSubcoreMesh(core_axis_name="c", subcore_axis_name="s")

    @functools.partial(
        pl.kernel,
        mesh=mesh,
        out_type=jax.ShapeDtypeStruct((P, H), jnp.float32),
        scratch_types=[
            pltpu.VMEM((_TPW,), jnp.int32),
            pltpu.VMEM((_TPW, H), jnp.float32),
            pltpu.SemaphoreType.DMA,
        ],
    )
    def _dispatch(x_hbm, pos_hbm, xs_hbm, idx_v, rows_v, sem):
        wid = lax.axis_index("s") * _NC + lax.axis_index("c")
        base = wid * _TPW
        pltpu.sync_copy(pos_hbm.at[pl.ds(base, _TPW)], idx_v)
        pltpu.sync_copy(x_hbm.at[pl.ds(base, _TPW)], rows_v)
        pltpu.async_copy(rows_v, xs_hbm.at[idx_v], sem).wait()

    return _dispatch


def _issue_fetch(e, slot, w1_hbm, w3_hbm, w2_hbm, w1b, w3b, w2b, sems):
    pltpu.make_async_copy(w1_hbm.at[e], w1b.at[slot], sems.at[slot, 0]).start()
    pltpu.make_async_copy(w3_hbm.at[e], w3b.at[slot], sems.at[slot, 1]).start()
    pltpu.make_async_copy(w2_hbm.at[e], w2b.at[slot], sems.at[slot, 2]).start()


def _ffn_body(bk_ref, xs_ref, w1_hbm, w3_hbm, w2_hbm, os_ref,
              w1b, w3b, w2b, sems):
    i = pl.program_id(0)
    act = bk_ref[i, 2]
    slot = bk_ref[i, 3]

    @pl.when(i == 0)
    def _():
        _issue_fetch(bk_ref[0, 0], 0, w1_hbm, w3_hbm, w2_hbm,
                     w1b, w3b, w2b, sems)

        @pl.when(bk_ref[0, 9] == 1)
        def _():
            _issue_fetch(bk_ref[0, 8], 1, w1_hbm, w3_hbm, w2_hbm,
                         w1b, w3b, w2b, sems)

    @pl.when((act == 1) & (bk_ref[i, 4] == 1))
    def _():
        pltpu.make_async_copy(w1_hbm.at[0], w1b.at[slot], sems.at[slot, 0]).wait()
        pltpu.make_async_copy(w3_hbm.at[0], w3b.at[slot], sems.at[slot, 1]).wait()
        pltpu.make_async_copy(w2_hbm.at[0], w2b.at[slot], sems.at[slot, 2]).wait()

    @pl.when(act == 1)
    def _():
        x = xs_ref[...]
        w1c = w1b[pl.ds(slot, 1)][0]
        w3c = w3b[pl.ds(slot, 1)][0]
        w2c = w2b[pl.ds(slot, 1)][0]
        a = jnp.dot(x, w1c, preferred_element_type=jnp.float32)
        b = jnp.dot(x, w3c, preferred_element_type=jnp.float32)
        h = (a * jax.nn.sigmoid(a)) * b
        os_ref[...] = jnp.dot(h, w2c, preferred_element_type=jnp.float32)

    @pl.when((act == 1) & (bk_ref[i, 5] == 1) & (bk_ref[i, 7] == 1))
    def _():
        _issue_fetch(bk_ref[i, 6], slot, w1_hbm, w3_hbm, w2_hbm,
                     w1b, w3b, w2b, sems)


def _ffn(bk, xs, w1, w3, w2):
    grid_spec = pltpu.PrefetchScalarGridSpec(
        num_scalar_prefetch=1,
        grid=(NT,),
        in_specs=[
            pl.BlockSpec((M, H), lambda i, bk: (bk[i, 1], 0)),
            pl.BlockSpec(memory_space=pl.ANY),
            pl.BlockSpec(memory_space=pl.ANY),
            pl.BlockSpec(memory_space=pl.ANY),
        ],
        out_specs=pl.BlockSpec((M, H), lambda i, bk: (bk[i, 1], 0)),
        scratch_shapes=[
            pltpu.VMEM((2, H, F), jnp.float32),
            pltpu.VMEM((2, H, F), jnp.float32),
            pltpu.VMEM((2, F, H), jnp.float32),
            pltpu.SemaphoreType.DMA((2, 3)),
        ],
    )
    return pl.pallas_call(
        _ffn_body,
        grid_spec=grid_spec,
        out_shape=jax.ShapeDtypeStruct((P, H), jnp.float32),
        compiler_params=pltpu.CompilerParams(
            dimension_semantics=("arbitrary",),
        ),
    )(bk, xs, w1, w3, w2)


@functools.cache
def _make_combine():
    mesh = plsc.VectorSubcoreMesh(core_axis_name="c", subcore_axis_name="s")

    @functools.partial(
        pl.kernel,
        mesh=mesh,
        out_type=jax.ShapeDtypeStruct((T, H), jnp.float32),
        scratch_types=[
            pltpu.VMEM((_TPW,), jnp.int32),
            pltpu.VMEM((_TPW, H), jnp.float32),
            pltpu.SemaphoreType.DMA,
        ],
    )
    def _combine(os_hbm, pos_hbm, out_hbm, idx_v, rows_v, sem):
        wid = lax.axis_index("s") * _NC + lax.axis_index("c")
        base = wid * _TPW
        pltpu.sync_copy(pos_hbm.at[pl.ds(base, _TPW)], idx_v)
        pltpu.async_copy(os_hbm.at[idx_v], rows_v, sem).wait()
        pltpu.sync_copy(rows_v, out_hbm.at[pl.ds(base, _TPW)])

    return _combine


def kernel(hidden_states, gate_w, w1, w2, w3):
    router_logits = hidden_states @ gate_w
    probs = jax.nn.softmax(router_logits, axis=-1)
    probs_pad = jnp.pad(probs, ((0, 0), (0, 128 - E)))
    pos2d, bk = _router(probs_pad)
    pos = pos2d.reshape(T)

    xs = _make_dispatch()(hidden_states, pos)
    os_ = _ffn(bk, xs, w1, w3, w2)
    final = _make_combine()(os_, pos)
    return (final, router_logits)
